# Initial kernel scaffold; baseline (speedup 1.0000x reference)
import jax
import jax.numpy as jnp
from jax.experimental import pallas as pl


def kernel(x, edge_index, W1, W2, eps):
    raise NotImplementedError("write your pallas kernel here")


# speedup vs baseline: 10.2314x; 10.2314x over previous
import functools

import jax
import jax.numpy as jnp
from jax import lax
from jax.experimental import pallas as pl
from jax.experimental.pallas import tpu as pltpu
from jax.experimental.pallas import tpu_sc as plsc

NC = 2
NS = 16
B = 128


def _round_up(a, b):
    return (a + b - 1) // b * b


def _make_edge_scatter(n_pad, dw, nbw):
    rpt = n_pad // NS
    mesh = plsc.VectorSubcoreMesh(core_axis_name="c", subcore_axis_name="s")

    @functools.partial(
        pl.kernel,
        out_type=jax.ShapeDtypeStruct((NC, n_pad, dw), jnp.float32),
        mesh=mesh,
        compiler_params=pltpu.CompilerParams(use_tc_tiling_on_sc=False),
        scratch_types=[
            pltpu.VMEM((nbw, B), jnp.int32),
            pltpu.VMEM((nbw, B), jnp.int32),
            pltpu.VMEM((B, dw), jnp.float32),
            pltpu.VMEM_SHARED((n_pad, dw), jnp.float32),
            pltpu.SemaphoreType.DMA,
        ],
    )
    def k(tbl, srcb, dstb, zeros, out, sidx, didx, rows, acc, sem):
        c = lax.axis_index("c")
        s = lax.axis_index("s")
        wid = c * NS + s
        pltpu.sync_copy(zeros.at[pl.ds(s * rpt, rpt)],
                        acc.at[pl.ds(s * rpt, rpt)])
        pltpu.sync_copy(srcb.at[pl.ds(wid * nbw, nbw)], sidx)
        pltpu.sync_copy(dstb.at[pl.ds(wid * nbw, nbw)], didx)
        plsc.subcore_barrier()

        def body(g, carry):
            pltpu.async_copy(tbl.at[sidx.at[g]], rows, sem).wait()
            pltpu.sync_copy(rows, acc.at[didx.at[g]], add=True)
            return carry

        lax.fori_loop(0, nbw, body, 0)
        plsc.subcore_barrier()
        pltpu.sync_copy(acc.at[pl.ds(s * rpt, rpt)],
                        out.at[c, pl.ds(s * rpt, rpt)])

    return k


def _make_deg_scatter(n_pad, dw, nbw):
    rpt = n_pad // NS
    mesh = plsc.VectorSubcoreMesh(core_axis_name="c", subcore_axis_name="s")

    @functools.partial(
        pl.kernel,
        out_type=jax.ShapeDtypeStruct((NC, n_pad, dw), jnp.float32),
        mesh=mesh,
        compiler_params=pltpu.CompilerParams(use_tc_tiling_on_sc=False),
        scratch_types=[
            pltpu.VMEM((nbw, B), jnp.int32),
            pltpu.VMEM((B, dw), jnp.float32),
            pltpu.VMEM_SHARED((n_pad, dw), jnp.float32),
        ],
    )
    def k(ones, dstb, zeros, out, didx, rows, acc):
        c = lax.axis_index("c")
        s = lax.axis_index("s")
        wid = c * NS + s
        pltpu.sync_copy(zeros.at[pl.ds(s * rpt, rpt)],
                        acc.at[pl.ds(s * rpt, rpt)])
        pltpu.sync_copy(dstb.at[pl.ds(wid * nbw, nbw)], didx)
        pltpu.sync_copy(ones, rows)
        plsc.subcore_barrier()

        def body(g, carry):
            pltpu.sync_copy(rows, acc.at[didx.at[g]], add=True)
            return carry

        lax.fori_loop(0, nbw, body, 0)
        plsc.subcore_barrier()
        pltpu.sync_copy(acc.at[pl.ds(s * rpt, rpt)],
                        out.at[c, pl.ds(s * rpt, rpt)])

    return k


def _dinv_block(d0, d1):
    return lax.rsqrt(d0[:, :1] + d1[:, :1] + 1.0)


def _k1_body(x_ref, w1_ref, d0_ref, d1_ref, o_ref):
    dinv = _dinv_block(d0_ref[...], d1_ref[...])
    h1 = jnp.dot(x_ref[...], w1_ref[...], preferred_element_type=jnp.float32)
    o_ref[...] = h1 * dinv


def _k3_body(p0_ref, p1_ref, h1p_ref, eps_ref, d0_ref, d1_ref, o_ref, *, latent):
    dinv = _dinv_block(d0_ref[...], d1_ref[...])
    s = dinv * (p0_ref[...] + p1_ref[...] + h1p_ref[...])
    mean = s[:, :latent]
    std = jax.nn.softplus(s[:, latent:]) + 1e-10
    z = mean + std * eps_ref[...]
    o_ref[...] = jnp.maximum(z, 0.0) * dinv


def _k5_body(q0_ref, q1_ref, hp_ref, w2_ref, d0_ref, d1_ref, o_ref):
    dinv = _dinv_block(d0_ref[...], d1_ref[...])
    t = dinv * (q0_ref[...] + q1_ref[...] + hp_ref[...])
    o_ref[...] = jnp.dot(t, w2_ref[...], preferred_element_type=jnp.float32)


def kernel(x, edge_index, W1, W2, eps):
    n, d = x.shape
    latent = eps.shape[1]
    d2 = W1.shape[1]
    e = edge_index.shape[1]

    n_pad = _round_up(n + 1, NS * 8)
    nb_total = _round_up(pl.cdiv(e, B), NC * NS * 8)
    nbw = nb_total // (NC * NS)
    e_pad = nb_total * B

    src = edge_index[0].astype(jnp.int32)
    dst = edge_index[1].astype(jnp.int32)
    src_b = jnp.concatenate(
        [src, jnp.zeros((e_pad - e,), jnp.int32)]).reshape(nb_total, B)
    dst_b = jnp.concatenate(
        [dst, jnp.full((e_pad - e,), n, jnp.int32)]).reshape(nb_total, B)

    zeros_w = jnp.zeros((n_pad, d2), jnp.float32)
    zeros_l = jnp.zeros((n_pad, latent), jnp.float32)
    zeros_16 = jnp.zeros((n_pad, 16), jnp.float32)
    ones_16 = jnp.ones((B, 16), jnp.float32)

    degp = _make_deg_scatter(n_pad, 16, nbw)(ones_16, dst_b, zeros_16)
    d0 = degp[0, :n, :]
    d1 = degp[1, :n, :]

    rb = 1000 if n % 1000 == 0 else 8
    grid = (n // rb,)
    h1p = pl.pallas_call(
        _k1_body,
        grid=grid,
        in_specs=[
            pl.BlockSpec((rb, d), lambda i: (i, 0)),
            pl.BlockSpec((d, d2), lambda i: (0, 0)),
            pl.BlockSpec((rb, 16), lambda i: (i, 0)),
            pl.BlockSpec((rb, 16), lambda i: (i, 0)),
        ],
        out_specs=pl.BlockSpec((rb, d2), lambda i: (i, 0)),
        out_shape=jax.ShapeDtypeStruct((n, d2), jnp.float32),
    )(x, W1, d0, d1)

    p = _make_edge_scatter(n_pad, d2, nbw)(h1p, src_b, dst_b, zeros_w)

    hp = pl.pallas_call(
        functools.partial(_k3_body, latent=latent),
        grid=grid,
        in_specs=[
            pl.BlockSpec((rb, d2), lambda i: (i, 0)),
            pl.BlockSpec((rb, d2), lambda i: (i, 0)),
            pl.BlockSpec((rb, d2), lambda i: (i, 0)),
            pl.BlockSpec((rb, latent), lambda i: (i, 0)),
            pl.BlockSpec((rb, 16), lambda i: (i, 0)),
            pl.BlockSpec((rb, 16), lambda i: (i, 0)),
        ],
        out_specs=pl.BlockSpec((rb, latent), lambda i: (i, 0)),
        out_shape=jax.ShapeDtypeStruct((n, latent), jnp.float32),
    )(p[0, :n, :], p[1, :n, :], h1p, eps, d0, d1)

    q = _make_edge_scatter(n_pad, latent, nbw)(hp, src_b, dst_b, zeros_l)

    out = pl.pallas_call(
        _k5_body,
        grid=grid,
        in_specs=[
            pl.BlockSpec((rb, latent), lambda i: (i, 0)),
            pl.BlockSpec((rb, latent), lambda i: (i, 0)),
            pl.BlockSpec((rb, latent), lambda i: (i, 0)),
            pl.BlockSpec((latent, d), lambda i: (0, 0)),
            pl.BlockSpec((rb, 16), lambda i: (i, 0)),
            pl.BlockSpec((rb, 16), lambda i: (i, 0)),
        ],
        out_specs=pl.BlockSpec((rb, d), lambda i: (i, 0)),
        out_shape=jax.ShapeDtypeStruct((n, d), jnp.float32),
    )(q[0, :n, :], q[1, :n, :], hp, W2, d0, d1)

    return out


# speedup vs baseline: 11.5838x; 1.1322x over previous
import functools

import jax
import jax.numpy as jnp
from jax import lax
from jax.experimental import pallas as pl
from jax.experimental.pallas import tpu as pltpu
from jax.experimental.pallas import tpu_sc as plsc

NC = 2
NS = 16
B = 128


def _round_up(a, b):
    return (a + b - 1) // b * b


def _make_edge_scatter(n_pad, dw, nbw):
    rpt = n_pad // NS
    mesh = plsc.VectorSubcoreMesh(core_axis_name="c", subcore_axis_name="s")

    @functools.partial(
        pl.kernel,
        out_type=jax.ShapeDtypeStruct((NC, n_pad, dw), jnp.float32),
        mesh=mesh,
        compiler_params=pltpu.CompilerParams(use_tc_tiling_on_sc=False),
        scratch_types=[
            pltpu.VMEM((2, B), jnp.int32),
            pltpu.VMEM((2, B), jnp.int32),
            pltpu.VMEM((B, dw), jnp.float32),
            pltpu.VMEM((B, dw), jnp.float32),
            pltpu.VMEM_SHARED((n_pad, dw), jnp.float32),
            pltpu.SemaphoreType.DMA,
            pltpu.SemaphoreType.DMA,
            pltpu.SemaphoreType.DMA,
            pltpu.SemaphoreType.DMA,
        ],
    )
    def k(tbl, idxb, zeros, out, ia, ib, r0, r1, acc, si0, si1, sr0, sr1):
        c = lax.axis_index("c")
        s = lax.axis_index("s")
        base = (c * NS + s) * nbw
        pltpu.sync_copy(zeros.at[pl.ds(s * rpt, rpt)],
                        acc.at[pl.ds(s * rpt, rpt)])
        plsc.subcore_barrier()

        pltpu.async_copy(idxb.at[base], ia, si0)
        pltpu.async_copy(idxb.at[base + 1], ib, si1)
        pltpu.make_async_copy(idxb.at[base], ia, si0).wait()
        pltpu.async_copy(tbl.at[ia.at[0]], r0, sr0)

        def body(i, carry):
            g2 = jnp.minimum(2 * i + 2, nbw - 1)
            g3 = jnp.minimum(2 * i + 3, nbw - 1)
            pltpu.make_async_copy(idxb.at[base], ib, si1).wait()
            pltpu.async_copy(tbl.at[ib.at[0]], r1, sr1)
            pltpu.make_async_copy(tbl.at[ia.at[0]], r0, sr0).wait()
            pltpu.sync_copy(r0, acc.at[ia.at[1]], add=True)
            pltpu.async_copy(idxb.at[base + g2], ia, si0)
            pltpu.make_async_copy(idxb.at[base], ia, si0).wait()
            pltpu.async_copy(tbl.at[ia.at[0]], r0, sr0)
            pltpu.make_async_copy(tbl.at[ib.at[0]], r1, sr1).wait()
            pltpu.sync_copy(r1, acc.at[ib.at[1]], add=True)
            pltpu.async_copy(idxb.at[base + g3], ib, si1)
            return carry

        lax.fori_loop(0, nbw // 2, body, 0)
        pltpu.make_async_copy(tbl.at[ia.at[0]], r0, sr0).wait()
        pltpu.make_async_copy(idxb.at[base], ib, si1).wait()
        plsc.subcore_barrier()
        pltpu.sync_copy(acc.at[pl.ds(s * rpt, rpt)],
                        out.at[c, pl.ds(s * rpt, rpt)])

    return k


def _make_deg_scatter(n_pad, dw, nbw):
    rpt = n_pad // NS
    mesh = plsc.VectorSubcoreMesh(core_axis_name="c", subcore_axis_name="s")

    @functools.partial(
        pl.kernel,
        out_type=jax.ShapeDtypeStruct((NC, n_pad, dw), jnp.float32),
        mesh=mesh,
        compiler_params=pltpu.CompilerParams(use_tc_tiling_on_sc=False),
        scratch_types=[
            pltpu.VMEM((nbw, B), jnp.int32),
            pltpu.VMEM((B, dw), jnp.float32),
            pltpu.VMEM_SHARED((n_pad, dw), jnp.float32),
        ],
    )
    def k(ones, dstb, zeros, out, didx, rows, acc):
        c = lax.axis_index("c")
        s = lax.axis_index("s")
        wid = c * NS + s
        pltpu.sync_copy(zeros.at[pl.ds(s * rpt, rpt)],
                        acc.at[pl.ds(s * rpt, rpt)])
        pltpu.sync_copy(dstb.at[pl.ds(wid * nbw, nbw)], didx)
        pltpu.sync_copy(ones, rows)
        plsc.subcore_barrier()

        def body(g, carry):
            pltpu.sync_copy(rows, acc.at[didx.at[g]], add=True)
            return carry

        lax.fori_loop(0, nbw, body, 0)
        plsc.subcore_barrier()
        pltpu.sync_copy(acc.at[pl.ds(s * rpt, rpt)],
                        out.at[c, pl.ds(s * rpt, rpt)])

    return k


def _dinv_block(d0, d1):
    return lax.rsqrt(d0[:, :1] + d1[:, :1] + 1.0)


def _k1_body(x_ref, w1_ref, d0_ref, d1_ref, o_ref):
    dinv = _dinv_block(d0_ref[...], d1_ref[...])
    h1 = jnp.dot(x_ref[...], w1_ref[...], preferred_element_type=jnp.float32)
    o_ref[...] = h1 * dinv


def _k3_body(p0_ref, p1_ref, h1p_ref, eps_ref, d0_ref, d1_ref, o_ref, *, latent):
    dinv = _dinv_block(d0_ref[...], d1_ref[...])
    s = dinv * (p0_ref[...] + p1_ref[...] + h1p_ref[...])
    mean = s[:, :latent]
    std = jax.nn.softplus(s[:, latent:]) + 1e-10
    z = mean + std * eps_ref[...]
    o_ref[...] = jnp.maximum(z, 0.0) * dinv


def _k5_body(q0_ref, q1_ref, hp_ref, w2_ref, d0_ref, d1_ref, o_ref):
    dinv = _dinv_block(d0_ref[...], d1_ref[...])
    t = dinv * (q0_ref[...] + q1_ref[...] + hp_ref[...])
    o_ref[...] = jnp.dot(t, w2_ref[...], preferred_element_type=jnp.float32)


def kernel(x, edge_index, W1, W2, eps):
    n, d = x.shape
    latent = eps.shape[1]
    d2 = W1.shape[1]
    e = edge_index.shape[1]

    n_pad = _round_up(n + 1, NS * 8)
    nb_total = _round_up(pl.cdiv(e, B), NC * NS * 8)
    nbw = nb_total // (NC * NS)
    e_pad = nb_total * B

    src = edge_index[0].astype(jnp.int32)
    dst = edge_index[1].astype(jnp.int32)
    src_b = jnp.concatenate(
        [src, jnp.zeros((e_pad - e,), jnp.int32)]).reshape(nb_total, B)
    pad_dst = n + jnp.arange(e_pad - e, dtype=jnp.int32) % (n_pad - n)
    dst_b = jnp.concatenate([dst, pad_dst]).reshape(nb_total, B)
    idx_b = jnp.stack([src_b, dst_b], axis=1)

    zeros_w = jnp.zeros((n_pad, d2), jnp.float32)
    zeros_l = jnp.zeros((n_pad, latent), jnp.float32)
    zeros_16 = jnp.zeros((n_pad, 16), jnp.float32)
    ones_16 = jnp.ones((B, 16), jnp.float32)

    degp = _make_deg_scatter(n_pad, 16, nbw)(ones_16, dst_b, zeros_16)
    d0 = degp[0, :n, :]
    d1 = degp[1, :n, :]

    rb = 1000 if n % 1000 == 0 else 8
    grid = (n // rb,)
    h1p = pl.pallas_call(
        _k1_body,
        grid=grid,
        in_specs=[
            pl.BlockSpec((rb, d), lambda i: (i, 0)),
            pl.BlockSpec((d, d2), lambda i: (0, 0)),
            pl.BlockSpec((rb, 16), lambda i: (i, 0)),
            pl.BlockSpec((rb, 16), lambda i: (i, 0)),
        ],
        out_specs=pl.BlockSpec((rb, d2), lambda i: (i, 0)),
        out_shape=jax.ShapeDtypeStruct((n, d2), jnp.float32),
    )(x, W1, d0, d1)

    p = _make_edge_scatter(n_pad, d2, nbw)(h1p, idx_b, zeros_w)

    hp = pl.pallas_call(
        functools.partial(_k3_body, latent=latent),
        grid=grid,
        in_specs=[
            pl.BlockSpec((rb, d2), lambda i: (i, 0)),
            pl.BlockSpec((rb, d2), lambda i: (i, 0)),
            pl.BlockSpec((rb, d2), lambda i: (i, 0)),
            pl.BlockSpec((rb, latent), lambda i: (i, 0)),
            pl.BlockSpec((rb, 16), lambda i: (i, 0)),
            pl.BlockSpec((rb, 16), lambda i: (i, 0)),
        ],
        out_specs=pl.BlockSpec((rb, latent), lambda i: (i, 0)),
        out_shape=jax.ShapeDtypeStruct((n, latent), jnp.float32),
    )(p[0, :n, :], p[1, :n, :], h1p, eps, d0, d1)

    q = _make_edge_scatter(n_pad, latent, nbw)(hp, idx_b, zeros_l)

    out = pl.pallas_call(
        _k5_body,
        grid=grid,
        in_specs=[
            pl.BlockSpec((rb, latent), lambda i: (i, 0)),
            pl.BlockSpec((rb, latent), lambda i: (i, 0)),
            pl.BlockSpec((rb, latent), lambda i: (i, 0)),
            pl.BlockSpec((latent, d), lambda i: (0, 0)),
            pl.BlockSpec((rb, 16), lambda i: (i, 0)),
            pl.BlockSpec((rb, 16), lambda i: (i, 0)),
        ],
        out_specs=pl.BlockSpec((rb, d), lambda i: (i, 0)),
        out_shape=jax.ShapeDtypeStruct((n, d), jnp.float32),
    )(q[0, :n, :], q[1, :n, :], hp, W2, d0, d1)

    return out


# speedup vs baseline: 31.8190x; 2.7469x over previous
import functools

import jax
import jax.numpy as jnp
from jax import lax
from jax.experimental import pallas as pl
from jax.experimental.pallas import tpu as pltpu
from jax.experimental.pallas import tpu_sc as plsc

NC = 2
NS = 16
B = 128


def _round_up(a, b):
    return (a + b - 1) // b * b


def _make_edge_scatter(n_pad, dw, nbw):
    rpt = n_pad // NS
    mesh = plsc.VectorSubcoreMesh(core_axis_name="c", subcore_axis_name="s")

    @functools.partial(
        pl.kernel,
        out_type=jax.ShapeDtypeStruct((NC, n_pad, dw), jnp.float32),
        mesh=mesh,
        compiler_params=pltpu.CompilerParams(use_tc_tiling_on_sc=False),
        scratch_types=[
            [pltpu.VMEM((2, B), jnp.int32) for _ in range(4)],
            pltpu.VMEM((B, dw), jnp.float32),
            pltpu.VMEM((B, dw), jnp.float32),
            pltpu.VMEM_SHARED((n_pad, dw), jnp.float32),
            [pltpu.SemaphoreType.DMA for _ in range(4)],
            pltpu.SemaphoreType.DMA,
            pltpu.SemaphoreType.DMA,
        ],
    )
    def k(tbl, idxb, zeros, out, ibufs, r0, r1, acc, sis, sr0, sr1):
        c = lax.axis_index("c")
        s = lax.axis_index("s")
        base = (c * NS + s) * nbw
        pltpu.sync_copy(zeros.at[pl.ds(s * rpt, rpt)],
                        acc.at[pl.ds(s * rpt, rpt)])
        plsc.subcore_barrier()

        rbufs = (r0, r1)
        srs = (sr0, sr1)
        for j in range(4):
            pltpu.async_copy(idxb.at[base + j], ibufs[j], sis[j])
        pltpu.make_async_copy(idxb.at[base], ibufs[0], sis[0]).wait()
        pltpu.async_copy(tbl.at[ibufs[0].at[0]], r0, sr0)

        def body(i, carry):
            g = 4 * i
            for j in range(4):
                inxt = ibufs[(j + 1) % 4]
                rcur = rbufs[j % 2]
                rnxt = rbufs[(j + 1) % 2]
                pltpu.make_async_copy(idxb.at[base], inxt, sis[(j + 1) % 4]).wait()
                pltpu.async_copy(tbl.at[inxt.at[0]], rnxt, srs[(j + 1) % 2])
                pltpu.make_async_copy(tbl.at[inxt.at[0]], rcur, srs[j % 2]).wait()
                pltpu.sync_copy(rcur, acc.at[ibufs[j].at[1]], add=True)
                gpre = jnp.minimum(g + j + 4, nbw - 1)
                pltpu.async_copy(idxb.at[base + gpre], ibufs[j], sis[j])
            return carry

        lax.fori_loop(0, nbw // 4, body, 0)
        pltpu.make_async_copy(tbl.at[ibufs[0].at[0]], r0, sr0).wait()
        for j in range(1, 4):
            pltpu.make_async_copy(idxb.at[base], ibufs[j], sis[j]).wait()
        plsc.subcore_barrier()
        pltpu.sync_copy(acc.at[pl.ds(s * rpt, rpt)],
                        out.at[c, pl.ds(s * rpt, rpt)])

    return k


def _make_deg_scatter(n_pad, dw, nbw):
    rpt = n_pad // NS
    mesh = plsc.VectorSubcoreMesh(core_axis_name="c", subcore_axis_name="s")

    @functools.partial(
        pl.kernel,
        out_type=jax.ShapeDtypeStruct((NC, n_pad, dw), jnp.float32),
        mesh=mesh,
        compiler_params=pltpu.CompilerParams(use_tc_tiling_on_sc=False),
        scratch_types=[
            pltpu.VMEM((nbw, B), jnp.int32),
            pltpu.VMEM((B, dw), jnp.float32),
            pltpu.VMEM_SHARED((n_pad, dw), jnp.float32),
        ],
    )
    def k(ones, dstb, zeros, out, didx, rows, acc):
        c = lax.axis_index("c")
        s = lax.axis_index("s")
        wid = c * NS + s
        pltpu.sync_copy(zeros.at[pl.ds(s * rpt, rpt)],
                        acc.at[pl.ds(s * rpt, rpt)])
        pltpu.sync_copy(dstb.at[pl.ds(wid * nbw, nbw)], didx)
        pltpu.sync_copy(ones, rows)
        plsc.subcore_barrier()

        def body(g, carry):
            pltpu.sync_copy(rows, acc.at[didx.at[g]], add=True)
            return carry

        lax.fori_loop(0, nbw, body, 0)
        plsc.subcore_barrier()
        pltpu.sync_copy(acc.at[pl.ds(s * rpt, rpt)],
                        out.at[c, pl.ds(s * rpt, rpt)])

    return k


def _dinv_block(d0, d1):
    return lax.rsqrt(d0[:, :1] + d1[:, :1] + 1.0)


def _k1_body(x_ref, w1_ref, d0_ref, d1_ref, o_ref):
    dinv = _dinv_block(d0_ref[...], d1_ref[...])
    h1 = jnp.dot(x_ref[...], w1_ref[...], preferred_element_type=jnp.float32)
    o_ref[...] = h1 * dinv


def _k3_body(p0_ref, p1_ref, h1p_ref, eps_ref, d0_ref, d1_ref, o_ref, *, latent):
    dinv = _dinv_block(d0_ref[...], d1_ref[...])
    s = dinv * (p0_ref[...] + p1_ref[...] + h1p_ref[...])
    mean = s[:, :latent]
    std = jax.nn.softplus(s[:, latent:]) + 1e-10
    z = mean + std * eps_ref[...]
    o_ref[...] = jnp.maximum(z, 0.0) * dinv


def _k5_body(q0_ref, q1_ref, hp_ref, w2_ref, d0_ref, d1_ref, o_ref):
    dinv = _dinv_block(d0_ref[...], d1_ref[...])
    t = dinv * (q0_ref[...] + q1_ref[...] + hp_ref[...])
    o_ref[...] = jnp.dot(t, w2_ref[...], preferred_element_type=jnp.float32)


def kernel(x, edge_index, W1, W2, eps):
    n, d = x.shape
    latent = eps.shape[1]
    d2 = W1.shape[1]
    e = edge_index.shape[1]

    n_pad = _round_up(n + 1, NS * 8)
    nb_total = _round_up(pl.cdiv(e, B), NC * NS * 8)
    nbw = nb_total // (NC * NS)
    e_pad = nb_total * B

    src = edge_index[0].astype(jnp.int32)
    dst = edge_index[1].astype(jnp.int32)
    pad_src = jnp.arange(e_pad - e, dtype=jnp.int32) % n
    src_b = jnp.concatenate([src, pad_src]).reshape(nb_total, B)
    pad_dst = n + jnp.arange(e_pad - e, dtype=jnp.int32) % (n_pad - n)
    dst_b = jnp.concatenate([dst, pad_dst]).reshape(nb_total, B)
    idx_b = jnp.stack([src_b, dst_b], axis=1)

    zeros_w = jnp.zeros((n_pad, d2), jnp.float32)
    zeros_l = jnp.zeros((n_pad, latent), jnp.float32)
    zeros_16 = jnp.zeros((n_pad, 16), jnp.float32)
    ones_16 = jnp.ones((B, 16), jnp.float32)

    degp = _make_deg_scatter(n_pad, 16, nbw)(ones_16, dst_b, zeros_16)
    d0 = degp[0, :n, :]
    d1 = degp[1, :n, :]

    rb = 1000 if n % 1000 == 0 else 8
    grid = (n // rb,)
    h1p = pl.pallas_call(
        _k1_body,
        grid=grid,
        in_specs=[
            pl.BlockSpec((rb, d), lambda i: (i, 0)),
            pl.BlockSpec((d, d2), lambda i: (0, 0)),
            pl.BlockSpec((rb, 16), lambda i: (i, 0)),
            pl.BlockSpec((rb, 16), lambda i: (i, 0)),
        ],
        out_specs=pl.BlockSpec((rb, d2), lambda i: (i, 0)),
        out_shape=jax.ShapeDtypeStruct((n, d2), jnp.float32),
    )(x, W1, d0, d1)

    p = _make_edge_scatter(n_pad, d2, nbw)(h1p, idx_b, zeros_w)

    hp = pl.pallas_call(
        functools.partial(_k3_body, latent=latent),
        grid=grid,
        in_specs=[
            pl.BlockSpec((rb, d2), lambda i: (i, 0)),
            pl.BlockSpec((rb, d2), lambda i: (i, 0)),
            pl.BlockSpec((rb, d2), lambda i: (i, 0)),
            pl.BlockSpec((rb, latent), lambda i: (i, 0)),
            pl.BlockSpec((rb, 16), lambda i: (i, 0)),
            pl.BlockSpec((rb, 16), lambda i: (i, 0)),
        ],
        out_specs=pl.BlockSpec((rb, latent), lambda i: (i, 0)),
        out_shape=jax.ShapeDtypeStruct((n, latent), jnp.float32),
    )(p[0, :n, :], p[1, :n, :], h1p, eps, d0, d1)

    q = _make_edge_scatter(n_pad, latent, nbw)(hp, idx_b, zeros_l)

    out = pl.pallas_call(
        _k5_body,
        grid=grid,
        in_specs=[
            pl.BlockSpec((rb, latent), lambda i: (i, 0)),
            pl.BlockSpec((rb, latent), lambda i: (i, 0)),
            pl.BlockSpec((rb, latent), lambda i: (i, 0)),
            pl.BlockSpec((latent, d), lambda i: (0, 0)),
            pl.BlockSpec((rb, 16), lambda i: (i, 0)),
            pl.BlockSpec((rb, 16), lambda i: (i, 0)),
        ],
        out_specs=pl.BlockSpec((rb, d), lambda i: (i, 0)),
        out_shape=jax.ShapeDtypeStruct((n, d), jnp.float32),
    )(q[0, :n, :], q[1, :n, :], hp, W2, d0, d1)

    return out
